# baseline (speedup 1.0000x reference)
import functools

import jax
import jax.numpy as jnp
from jax import lax
from jax.experimental import pallas as pl
from jax.experimental.pallas import tpu as pltpu
from jax.experimental.pallas import tpu_sc as plsc

_LANES = 16


@functools.cache
def _make_gather(R: int, S: int, D: int):
    scale = float(D) ** 0.5
    info = plsc.get_sparse_core_info()
    nw = info.num_cores * info.num_subcores
    r_per_w = R // nw
    NB = 4
    n_chunks = r_per_w // NB
    assert r_per_w % NB == 0 and R % nw == 0

    mesh = plsc.VectorSubcoreMesh(core_axis_name="c", subcore_axis_name="s")

    @functools.partial(
        pl.kernel,
        mesh=mesh,
        out_type=jax.ShapeDtypeStruct((R, S, D), jnp.float32),
        scratch_types=[
            pltpu.VMEM((NB * S,), jnp.int32),
            pltpu.VMEM((NB, S, D), jnp.float32),
            pltpu.SemaphoreType.DMA,
        ],
        compiler_params=pltpu.CompilerParams(use_tc_tiling_on_sc=False),
    )
    def gather_kernel(idx_hbm, table_hbm, out_hbm, idx_v, rows_v, sem):
        wid = lax.axis_index("s") * info.num_cores + lax.axis_index("c")
        row0 = wid * r_per_w

        def chunk_body(ci, carry):
            br = row0 + ci * NB
            pltpu.sync_copy(idx_hbm.at[pl.ds(br * S, NB * S)], idx_v)
            for bi in range(NB):
                pltpu.async_copy(
                    table_hbm.at[idx_v.at[pl.ds(bi * S, S)]],
                    rows_v.at[bi],
                    sem,
                ).wait()

            def scale_body(r, c2):
                for bi in range(NB):
                    for j in range(D // _LANES):
                        sl = pl.ds(j * _LANES, _LANES)
                        rows_v[bi, r, sl] = rows_v[bi, r, sl] * scale
                return c2

            lax.fori_loop(0, S, scale_body, 0)
            pltpu.sync_copy(rows_v, out_hbm.at[pl.ds(br, NB)])
            return carry

        lax.fori_loop(0, n_chunks, chunk_body, 0)

    return gather_kernel


def kernel(x, table):
    R, S = x.shape
    D = table.shape[1]
    xf = x.reshape(R * S)
    return _make_gather(R, S, D)(xf, table)
